# baseline (speedup 1.0000x reference)
import functools

import jax
import jax.numpy as jnp
from jax import lax
from jax.experimental import pallas as pl
from jax.experimental.pallas import tpu as pltpu
from jax.experimental.pallas import tpu_sc as plsc

ACTIONS_MAX = 1000
EMB_SIZE = 128
NUM_AGENTS = 26
MLP_OUT = 1024
BATCH = 16384

DW = EMB_SIZE // 2
GROUP = 128
TOTAL_IDX = BATCH * NUM_AGENTS
NUM_GROUPS = TOTAL_IDX // GROUP

BM = 512


def _sc_gather(idx, tab_i32):
    info = plsc.get_sparse_core_info()
    nc, ns = info.num_cores, info.num_subcores
    nw = nc * ns
    mesh = plsc.VectorSubcoreMesh(core_axis_name="c", subcore_axis_name="s")

    @functools.partial(
        pl.kernel,
        mesh=mesh,
        out_type=jax.ShapeDtypeStruct((TOTAL_IDX, DW), jnp.int32),
        scratch_types=[
            pltpu.VMEM((GROUP,), jnp.int32),
            pltpu.VMEM((GROUP, DW), jnp.int32),
            pltpu.SemaphoreType.DMA,
        ],
        compiler_params=pltpu.CompilerParams(use_tc_tiling_on_sc=False),
    )
    def k(idx_hbm, tab_hbm, out_hbm, idx_v, rows_v, sem):
        wid = lax.axis_index("s") * nc + lax.axis_index("c")
        n_i = (NUM_GROUPS - wid + nw - 1) // nw

        def body(i, carry):
            base = (wid + i * nw) * GROUP
            pltpu.sync_copy(idx_hbm.at[pl.ds(base, GROUP)], idx_v)
            pltpu.async_copy(tab_hbm.at[idx_v], rows_v, sem).wait()
            pltpu.sync_copy(rows_v, out_hbm.at[pl.ds(base, GROUP)])
            return carry

        lax.fori_loop(0, n_i, body, 0)

    return k(idx, tab_i32)


def _tc_matmul(x, w, b2):
    k_dim = x.shape[1]

    def mm(x_ref, w_ref, b_ref, o_ref):
        acc = jnp.dot(x_ref[...], w_ref[...],
                      preferred_element_type=jnp.float32)
        o_ref[...] = acc + b_ref[...]

    return pl.pallas_call(
        mm,
        grid=(BATCH // BM,),
        in_specs=[
            pl.BlockSpec((BM, k_dim), lambda i: (i, 0)),
            pl.BlockSpec((k_dim, MLP_OUT), lambda i: (0, 0)),
            pl.BlockSpec((1, MLP_OUT), lambda i: (0, 0)),
        ],
        out_specs=pl.BlockSpec((BM, MLP_OUT), lambda i: (i, 0)),
        out_shape=jax.ShapeDtypeStruct((BATCH, MLP_OUT), jnp.float32),
    )(x, w, b2)


def kernel(discrete_actions, emb_table, W, b):
    idx = discrete_actions.reshape(-1).astype(jnp.int32)
    tab_bf = emb_table.astype(jnp.bfloat16)
    tab_i32 = lax.bitcast_convert_type(
        tab_bf.reshape(ACTIONS_MAX, DW, 2), jnp.int32)
    gathered = _sc_gather(idx, tab_i32)
    x_bf = lax.bitcast_convert_type(gathered, jnp.bfloat16).reshape(
        BATCH, NUM_AGENTS * EMB_SIZE)
    w_bf = W.astype(jnp.bfloat16)
    return _tc_matmul(x_bf, w_bf, b.reshape(1, MLP_OUT))


# speedup vs baseline: 48.5028x; 48.5028x over previous
import functools

import jax
import jax.numpy as jnp
from jax import lax
from jax.experimental import pallas as pl
from jax.experimental.pallas import tpu as pltpu
from jax.experimental.pallas import tpu_sc as plsc

ACTIONS_MAX = 1000
EMB_SIZE = 128
NUM_AGENTS = 26
MLP_OUT = 1024
BATCH = 16384

GROUP = 128
TOTAL_IDX = BATCH * NUM_AGENTS
NUM_GROUPS = TOTAL_IDX // GROUP

BM = 512


def _sc_gather(idx, tab):
    info = plsc.get_sparse_core_info()
    nc, ns = info.num_cores, info.num_subcores
    nw = nc * ns
    mesh = plsc.VectorSubcoreMesh(core_axis_name="c", subcore_axis_name="s")

    @functools.partial(
        pl.kernel,
        mesh=mesh,
        out_type=jax.ShapeDtypeStruct((TOTAL_IDX, EMB_SIZE), jnp.float32),
        scratch_types=[
            pltpu.VMEM((GROUP,), jnp.int32),
            pltpu.VMEM((GROUP, EMB_SIZE), jnp.float32),
            pltpu.SemaphoreType.DMA,
        ],
    )
    def k(idx_hbm, tab_hbm, out_hbm, idx_v, rows_v, sem):
        wid = lax.axis_index("s") * nc + lax.axis_index("c")
        n_i = (NUM_GROUPS - wid + nw - 1) // nw

        def body(i, carry):
            base = (wid + i * nw) * GROUP
            pltpu.sync_copy(idx_hbm.at[pl.ds(base, GROUP)], idx_v)
            pltpu.async_copy(tab_hbm.at[idx_v], rows_v, sem).wait()
            pltpu.sync_copy(rows_v, out_hbm.at[pl.ds(base, GROUP)])
            return carry

        lax.fori_loop(0, n_i, body, 0)

    return k(idx, tab)


def _tc_matmul(x, w, b2):
    k_dim = x.shape[1]

    def mm(x_ref, w_ref, b_ref, o_ref):
        acc = jnp.dot(x_ref[...].astype(jnp.bfloat16), w_ref[...],
                      preferred_element_type=jnp.float32)
        o_ref[...] = acc + b_ref[...]

    return pl.pallas_call(
        mm,
        grid=(BATCH // BM,),
        in_specs=[
            pl.BlockSpec((BM, k_dim), lambda i: (i, 0)),
            pl.BlockSpec((k_dim, MLP_OUT), lambda i: (0, 0)),
            pl.BlockSpec((1, MLP_OUT), lambda i: (0, 0)),
        ],
        out_specs=pl.BlockSpec((BM, MLP_OUT), lambda i: (i, 0)),
        out_shape=jax.ShapeDtypeStruct((BATCH, MLP_OUT), jnp.float32),
    )(x, w, b2)


def kernel(discrete_actions, emb_table, W, b):
    idx = discrete_actions.reshape(-1).astype(jnp.int32)
    gathered = _sc_gather(idx, emb_table)
    x = gathered.reshape(BATCH, NUM_AGENTS * EMB_SIZE)
    w_bf = W.astype(jnp.bfloat16)
    return _tc_matmul(x, w_bf, b.reshape(1, MLP_OUT))


# speedup vs baseline: 53.0362x; 1.0935x over previous
import functools

import jax
import jax.numpy as jnp
from jax import lax
from jax.experimental import pallas as pl
from jax.experimental.pallas import tpu as pltpu
from jax.experimental.pallas import tpu_sc as plsc

ACTIONS_MAX = 1000
EMB_SIZE = 128
NUM_AGENTS = 26
MLP_OUT = 1024
BATCH = 16384

GROUP = 128
TOTAL_IDX = BATCH * NUM_AGENTS
NUM_GROUPS = TOTAL_IDX // GROUP

BM = 512


def _sc_gather(idx, tab):
    n_idx = idx.shape[0]
    num_groups = n_idx // GROUP
    info = plsc.get_sparse_core_info()
    nc, ns = info.num_cores, info.num_subcores
    nw = nc * ns
    mesh = plsc.VectorSubcoreMesh(core_axis_name="c", subcore_axis_name="s")

    @functools.partial(
        pl.kernel,
        mesh=mesh,
        out_type=jax.ShapeDtypeStruct((n_idx, EMB_SIZE), jnp.float32),
        scratch_types=[
            pltpu.VMEM((GROUP,), jnp.int32),
            pltpu.VMEM((GROUP, EMB_SIZE), jnp.float32),
            pltpu.SemaphoreType.DMA,
        ],
    )
    def k(idx_hbm, tab_hbm, out_hbm, idx_v, rows_v, sem):
        wid = lax.axis_index("s") * nc + lax.axis_index("c")
        n_i = (num_groups - wid + nw - 1) // nw

        def body(i, carry):
            base = (wid + i * nw) * GROUP
            pltpu.sync_copy(idx_hbm.at[pl.ds(base, GROUP)], idx_v)
            pltpu.async_copy(tab_hbm.at[idx_v], rows_v, sem).wait()
            pltpu.sync_copy(rows_v, out_hbm.at[pl.ds(base, GROUP)])
            return carry

        lax.fori_loop(0, n_i, body, 0)

    return k(idx, tab)


def _tc_matmul(x, w, b2):
    m, k_dim = x.shape

    def mm(x_ref, w_ref, b_ref, o_ref):
        acc = jnp.dot(x_ref[...].astype(jnp.bfloat16), w_ref[...],
                      preferred_element_type=jnp.float32)
        o_ref[...] = acc + b_ref[...]

    return pl.pallas_call(
        mm,
        grid=(m // BM,),
        in_specs=[
            pl.BlockSpec((BM, k_dim), lambda i: (i, 0)),
            pl.BlockSpec((k_dim, MLP_OUT), lambda i: (0, 0)),
            pl.BlockSpec((1, MLP_OUT), lambda i: (0, 0)),
        ],
        out_specs=pl.BlockSpec((BM, MLP_OUT), lambda i: (i, 0)),
        out_shape=jax.ShapeDtypeStruct((m, MLP_OUT), jnp.float32),
    )(x, w, b2)


NUM_CHUNKS = 4


def kernel(discrete_actions, emb_table, W, b):
    idx = discrete_actions.reshape(-1).astype(jnp.int32)
    w_bf = W.astype(jnp.bfloat16)
    b2 = b.reshape(1, MLP_OUT)
    bc = BATCH // NUM_CHUNKS
    outs = []
    for c in range(NUM_CHUNKS):
        idx_c = lax.dynamic_slice_in_dim(idx, c * bc * NUM_AGENTS,
                                         bc * NUM_AGENTS)
        g = _sc_gather(idx_c, emb_table)
        outs.append(_tc_matmul(g.reshape(bc, NUM_AGENTS * EMB_SIZE),
                               w_bf, b2))
    return jnp.concatenate(outs, axis=0)
